# baseline (speedup 1.0000x reference)
import functools

import jax
import jax.numpy as jnp
from jax import lax
from jax.experimental import pallas as pl
from jax.experimental.pallas import tpu as pltpu
from jax.experimental.pallas import tpu_sc as plsc

NUM_COLS = 26
VOCAB = 100000
EMBED_DIM = 32
BATCH = 16384

TOTAL = BATCH * NUM_COLS
NC, NS, LANES = 2, 16, 16
NW = NC * NS
PER_W = TOTAL // NW
SPW = 128
ROWS_W = PER_W // SPW
CHUNK = 1024
SPC = CHUNK // SPW
NCHUNK = PER_W // CHUNK


def _body(x_hbm, tab_hbm, out_hbm, idx_v, rows_v, gsem):
    wid = lax.axis_index("s") * NC + lax.axis_index("c")
    base_row = wid * ROWS_W

    pltpu.sync_copy(x_hbm.at[pl.ds(base_row, ROWS_W)], idx_v)

    lane = lax.iota(jnp.int32, LANES)

    def add_off(j, _):
        for k in range(SPW // LANES):
            pos = lane + (j * SPW + k * LANES)
            off = (pos % NUM_COLS) * VOCAB
            sl = pl.ds(k * LANES, LANES)
            idx_v[j, sl] = idx_v[j, sl] + off
        return 0

    lax.fori_loop(0, ROWS_W, add_off, 0)

    def chunk(c, _):
        cps = [
            pltpu.async_copy(
                tab_hbm.at[idx_v.at[c * SPC + s]],
                rows_v.at[pl.ds(s * SPW, SPW)],
                gsem,
            )
            for s in range(SPC)
        ]
        for cp in cps:
            cp.wait()
        pltpu.sync_copy(rows_v, out_hbm.at[pl.ds(wid * PER_W + c * CHUNK, CHUNK)])
        return 0

    lax.fori_loop(0, NCHUNK, chunk, 0)


@functools.partial(
    pl.kernel,
    out_type=jax.ShapeDtypeStruct((TOTAL, EMBED_DIM), jnp.float32),
    mesh=plsc.VectorSubcoreMesh(
        core_axis_name="c", subcore_axis_name="s", num_cores=NC, num_subcores=NS
    ),
    scratch_types=[
        pltpu.VMEM((ROWS_W, SPW), jnp.int32),
        pltpu.VMEM((CHUNK, EMBED_DIM), jnp.float32),
        pltpu.SemaphoreType.DMA,
    ],
    compiler_params=pltpu.CompilerParams(use_tc_tiling_on_sc=False),
)
def _gather(x_hbm, tab_hbm, out_hbm, idx_v, rows_v, gsem):
    _body(x_hbm, tab_hbm, out_hbm, idx_v, rows_v, gsem)


def kernel(x_batch, tables):
    x_flat = x_batch.astype(jnp.int32).reshape(TOTAL // SPW, SPW)
    tab = tables.reshape(NUM_COLS * VOCAB, EMBED_DIM)
    out = _gather(x_flat, tab)
    return out.reshape(BATCH, NUM_COLS * EMBED_DIM)
